# initial kernel scaffold (unmeasured)
import jax
import jax.numpy as jnp
from jax import lax
from jax.experimental import pallas as pl
from jax.experimental.pallas import tpu as pltpu

N_DEV = 4
B = 2
S_LOC = 128
S_GLOB = S_LOC * N_DEV
D = 512
HQ = 4
DH = 64
HD = HQ * DH


def kernel(x, Wq, Wk, Wv, Wo):
    def body(x_ref, wq_ref, wk_ref, wv_ref, wo_ref, out_ref,
             q_scr, k_full, v_full, comm, send_sems, recv_sems):
        my_pos = lax.axis_index("i")
        left = lax.rem(my_pos - 1 + N_DEV, N_DEV)
        right = lax.rem(my_pos + 1, N_DEV)

        barrier_sem = pltpu.get_barrier_semaphore()
        for nbr in [left, right]:
            pl.semaphore_signal(
                barrier_sem, inc=1,
                device_id=(nbr,), device_id_type=pl.DeviceIdType.MESH,
            )
        pl.semaphore_wait(barrier_sem, 2)

        pos = (my_pos * S_LOC).astype(jnp.float32) + lax.broadcasted_iota(
            jnp.float32, (S_LOC, HD), 0)
        lane = lax.broadcasted_iota(jnp.int32, (S_LOC, HD), 1)
        d = lane % DH
        expo = (d - (d % 2)).astype(jnp.float32) / DH
        inv = jnp.exp(-jnp.log(10000.0) * expo)
        ang = pos * inv
        cos = jnp.cos(ang)
        sin = jnp.sin(ang)

        r = lax.broadcasted_iota(jnp.int32, (HD, HD), 0)
        c = lax.broadcasted_iota(jnp.int32, (HD, HD), 1)
        P = (((c == r + 1) & (r % 2 == 0)).astype(jnp.float32)
             - ((c == r - 1) & (r % 2 == 1)).astype(jnp.float32))

        for b in range(B):
            xb = x_ref[b]
            qp = jnp.dot(xb, wq_ref[...], preferred_element_type=jnp.float32)
            q_scr[b] = qp * cos + jnp.dot(
                qp, P, preferred_element_type=jnp.float32) * sin
            kp = jnp.dot(xb, wk_ref[...], preferred_element_type=jnp.float32)
            kb = kp * cos + jnp.dot(
                kp, P, preferred_element_type=jnp.float32) * sin
            vb = jnp.dot(xb, wv_ref[...], preferred_element_type=jnp.float32)
            comm[0, 0, b] = kb
            comm[0, 1, b] = vb
            k_full[b, pl.ds(my_pos * S_LOC, S_LOC), :] = kb
            v_full[b, pl.ds(my_pos * S_LOC, S_LOC), :] = vb

        for h in range(N_DEV - 1):
            rdma = pltpu.make_async_remote_copy(
                src_ref=comm.at[h],
                dst_ref=comm.at[h + 1],
                send_sem=send_sems.at[h],
                recv_sem=recv_sems.at[h],
                device_id=(right,),
                device_id_type=pl.DeviceIdType.MESH,
            )
            rdma.start()
            rdma.wait()
            origin = lax.rem(my_pos - (h + 1) + N_DEV, N_DEV)
            for b in range(B):
                k_full[b, pl.ds(origin * S_LOC, S_LOC), :] = comm[h + 1, 0, b]
                v_full[b, pl.ds(origin * S_LOC, S_LOC), :] = comm[h + 1, 1, b]

        for b in range(B):
            ctx_heads = []
            for hh in range(HQ):
                qh = q_scr[b, :, hh * DH:(hh + 1) * DH]
                kh = k_full[b, :, hh * DH:(hh + 1) * DH]
                vh = v_full[b, :, hh * DH:(hh + 1) * DH]
                s = lax.dot_general(
                    qh, kh, (((1,), (1,)), ((), ())),
                    preferred_element_type=jnp.float32) * 0.125
                m = jnp.max(s, axis=-1, keepdims=True)
                w = jnp.exp(s - m)
                w = w / jnp.sum(w, axis=-1, keepdims=True)
                ctx_heads.append(jnp.dot(
                    w, vh, preferred_element_type=jnp.float32))
            ctx = jnp.concatenate(ctx_heads, axis=1)
            out_ref[b] = jnp.dot(
                ctx, wo_ref[...], preferred_element_type=jnp.float32)

    return pl.pallas_call(
        body,
        out_shape=jax.ShapeDtypeStruct((B, S_LOC, D), jnp.float32),
        in_specs=[pl.BlockSpec(memory_space=pltpu.VMEM)] * 5,
        out_specs=pl.BlockSpec(memory_space=pltpu.VMEM),
        scratch_shapes=[
            pltpu.VMEM((B, S_LOC, HD), jnp.float32),
            pltpu.VMEM((B, S_GLOB, HD), jnp.float32),
            pltpu.VMEM((B, S_GLOB, HD), jnp.float32),
            pltpu.VMEM((N_DEV, 2, B, S_LOC, HD), jnp.float32),
            pltpu.SemaphoreType.DMA((N_DEV - 1,)),
            pltpu.SemaphoreType.DMA((N_DEV - 1,)),
        ],
        compiler_params=pltpu.CompilerParams(collective_id=0),
    )(x, Wq, Wk, Wv, Wo)


# baseline (device time: 36217 ns/iter reference)
import jax
import jax.numpy as jnp
from jax import lax
from jax.experimental import pallas as pl
from jax.experimental.pallas import tpu as pltpu

N_DEV = 4
B = 2
S_LOC = 128
S_GLOB = S_LOC * N_DEV
D = 512
HQ = 4
DH = 64
HD = HQ * DH


def kernel(x, Wq, Wk, Wv, Wo):
    def body(x_ref, wq_ref, wk_ref, wv_ref, wo_ref, out_ref,
             q_scr, k_full, v_full, comm, send_sems, recv_sems):
        my_pos = lax.axis_index("i")
        left = lax.rem(my_pos - 1 + N_DEV, N_DEV)
        right = lax.rem(my_pos + 1, N_DEV)

        barrier_sem = pltpu.get_barrier_semaphore()
        for nbr in [left, right]:
            pl.semaphore_signal(
                barrier_sem, inc=1,
                device_id=(nbr,), device_id_type=pl.DeviceIdType.MESH,
            )
        pl.semaphore_wait(barrier_sem, 2)

        pos = (my_pos * S_LOC).astype(jnp.float32) + lax.broadcasted_iota(
            jnp.int32, (S_LOC, HD), 0).astype(jnp.float32)
        lane = lax.broadcasted_iota(jnp.int32, (S_LOC, HD), 1)
        d = lane % DH
        expo = (d - (d % 2)).astype(jnp.float32) / DH
        inv = jnp.exp(-jnp.log(10000.0) * expo)
        ang = pos * inv
        cos = jnp.cos(ang)
        sin = jnp.sin(ang)

        r = lax.broadcasted_iota(jnp.int32, (HD, HD), 0)
        c = lax.broadcasted_iota(jnp.int32, (HD, HD), 1)
        P = (((c == r + 1) & (r % 2 == 0)).astype(jnp.float32)
             - ((c == r - 1) & (r % 2 == 1)).astype(jnp.float32))

        for b in range(B):
            xb = x_ref[b]
            qp = jnp.dot(xb, wq_ref[...], preferred_element_type=jnp.float32)
            q_scr[b] = qp * cos + jnp.dot(
                qp, P, preferred_element_type=jnp.float32) * sin
            kp = jnp.dot(xb, wk_ref[...], preferred_element_type=jnp.float32)
            kb = kp * cos + jnp.dot(
                kp, P, preferred_element_type=jnp.float32) * sin
            vb = jnp.dot(xb, wv_ref[...], preferred_element_type=jnp.float32)
            comm[0, 0, b] = kb
            comm[0, 1, b] = vb
            k_full[b, pl.ds(my_pos * S_LOC, S_LOC), :] = kb
            v_full[b, pl.ds(my_pos * S_LOC, S_LOC), :] = vb

        for h in range(N_DEV - 1):
            rdma = pltpu.make_async_remote_copy(
                src_ref=comm.at[h],
                dst_ref=comm.at[h + 1],
                send_sem=send_sems.at[h],
                recv_sem=recv_sems.at[h],
                device_id=(right,),
                device_id_type=pl.DeviceIdType.MESH,
            )
            rdma.start()
            rdma.wait()
            origin = lax.rem(my_pos - (h + 1) + N_DEV, N_DEV)
            for b in range(B):
                k_full[b, pl.ds(origin * S_LOC, S_LOC), :] = comm[h + 1, 0, b]
                v_full[b, pl.ds(origin * S_LOC, S_LOC), :] = comm[h + 1, 1, b]

        for b in range(B):
            ctx_heads = []
            for hh in range(HQ):
                qh = q_scr[b, :, hh * DH:(hh + 1) * DH]
                kh = k_full[b, :, hh * DH:(hh + 1) * DH]
                vh = v_full[b, :, hh * DH:(hh + 1) * DH]
                s = lax.dot_general(
                    qh, kh, (((1,), (1,)), ((), ())),
                    preferred_element_type=jnp.float32) * 0.125
                m = jnp.max(s, axis=-1, keepdims=True)
                w = jnp.exp(s - m)
                w = w / jnp.sum(w, axis=-1, keepdims=True)
                ctx_heads.append(jnp.dot(
                    w, vh, preferred_element_type=jnp.float32))
            ctx = jnp.concatenate(ctx_heads, axis=1)
            out_ref[b] = jnp.dot(
                ctx, wo_ref[...], preferred_element_type=jnp.float32)

    return pl.pallas_call(
        body,
        out_shape=jax.ShapeDtypeStruct((B, S_LOC, D), jnp.float32),
        in_specs=[pl.BlockSpec(memory_space=pltpu.VMEM)] * 5,
        out_specs=pl.BlockSpec(memory_space=pltpu.VMEM),
        scratch_shapes=[
            pltpu.VMEM((B, S_LOC, HD), jnp.float32),
            pltpu.VMEM((B, S_GLOB, HD), jnp.float32),
            pltpu.VMEM((B, S_GLOB, HD), jnp.float32),
            pltpu.VMEM((N_DEV, 2, B, S_LOC, HD), jnp.float32),
            pltpu.SemaphoreType.DMA((N_DEV - 1,)),
            pltpu.SemaphoreType.DMA((N_DEV - 1,)),
        ],
        compiler_params=pltpu.CompilerParams(collective_id=0),
    )(x, Wq, Wk, Wv, Wo)


# device time: 25801 ns/iter; 1.4037x vs baseline; 1.4037x over previous
import jax
import jax.numpy as jnp
from jax import lax
from jax.experimental import pallas as pl
from jax.experimental.pallas import tpu as pltpu

N_DEV = 4
B = 2
S_LOC = 128
D = 512
HQ = 4
DH = 64
HD = HQ * DH
N_XFER = 12


def kernel(x, Wq, Wk, Wv, Wo):
    def body(x_ref, wq_ref, wk_ref, wv_ref, wo_ref, out_ref,
             q_scr, k_full, v_full, send_sems, recv_sems):
        my_pos = lax.axis_index("i")
        left = lax.rem(my_pos - 1 + N_DEV, N_DEV)
        right = lax.rem(my_pos + 1, N_DEV)
        far = lax.rem(my_pos + 2, N_DEV)

        barrier_sem = pltpu.get_barrier_semaphore()
        for nbr in [left, right]:
            pl.semaphore_signal(
                barrier_sem, inc=1,
                device_id=(nbr,), device_id_type=pl.DeviceIdType.MESH,
            )
        pl.semaphore_wait(barrier_sem, 2)

        def rdma(src, dst, idx, tgt):
            return pltpu.make_async_remote_copy(
                src_ref=src, dst_ref=dst,
                send_sem=send_sems.at[idx], recv_sem=recv_sems.at[idx],
                device_id=(tgt,), device_id_type=pl.DeviceIdType.MESH,
            )

        pos = (my_pos * S_LOC).astype(jnp.float32) + lax.broadcasted_iota(
            jnp.int32, (S_LOC, HD), 0).astype(jnp.float32)
        lane = lax.broadcasted_iota(jnp.int32, (S_LOC, HD), 1)
        d = lane % DH
        expo = (d - (d % 2)).astype(jnp.float32) / DH
        inv = jnp.exp(-jnp.log(10000.0) * expo)
        ang = pos * inv
        cos = jnp.cos(ang)
        sin = jnp.sin(ang)

        r = lax.broadcasted_iota(jnp.int32, (HD, HD), 0)
        c = lax.broadcasted_iota(jnp.int32, (HD, HD), 1)
        P = (((c == r + 1) & (r % 2 == 0)).astype(jnp.float32)
             - ((c == r - 1) & (r % 2 == 1)).astype(jnp.float32))

        sends = []
        for b in range(B):
            xb = x_ref[b]
            kp = jnp.dot(xb, wk_ref[...], preferred_element_type=jnp.float32)
            kb = kp * cos + jnp.dot(
                kp, P, preferred_element_type=jnp.float32) * sin
            vb = jnp.dot(xb, wv_ref[...], preferred_element_type=jnp.float32)
            k_full[b, my_pos] = kb
            v_full[b, my_pos] = vb
            for s in (
                rdma(k_full.at[b, my_pos], k_full.at[b, my_pos], 2 * b + 0, right),
                rdma(v_full.at[b, my_pos], v_full.at[b, my_pos], 2 * b + 1, right),
                rdma(k_full.at[b, my_pos], k_full.at[b, my_pos], 2 * b + 4, left),
                rdma(v_full.at[b, my_pos], v_full.at[b, my_pos], 2 * b + 5, left),
            ):
                s.start()
                sends.append(s)

        for b in range(B):
            qp = jnp.dot(x_ref[b], wq_ref[...],
                         preferred_element_type=jnp.float32)
            q_scr[b] = qp * cos + jnp.dot(
                qp, P, preferred_element_type=jnp.float32) * sin

        def recv(dst, idx):
            return rdma(dst, dst, idx, right)

        recvs = [
            recv(k_full.at[0, left], 0), recv(v_full.at[0, left], 1),
            recv(k_full.at[1, left], 2), recv(v_full.at[1, left], 3),
            recv(k_full.at[0, right], 4), recv(v_full.at[0, right], 5),
            recv(k_full.at[1, right], 6), recv(v_full.at[1, right], 7),
            recv(k_full.at[0, far], 8), recv(v_full.at[0, far], 9),
            recv(k_full.at[1, far], 10), recv(v_full.at[1, far], 11),
        ]

        recvs[0].wait_recv()
        recvs[1].wait_recv()
        for s in (rdma(k_full.at[0, left], k_full.at[0, left], 8, right),
                  rdma(v_full.at[0, left], v_full.at[0, left], 9, right)):
            s.start()
            sends.append(s)
        recvs[6].wait_recv()
        recvs[7].wait_recv()
        for s in (rdma(k_full.at[1, right], k_full.at[1, right], 10, left),
                  rdma(v_full.at[1, right], v_full.at[1, right], 11, left)):
            s.start()
            sends.append(s)

        for i in (2, 3, 4, 5, 8, 9, 10, 11):
            recvs[i].wait_recv()
        for s in sends:
            s.wait_send()

        for b in range(B):
            ctx_heads = []
            for hh in range(HQ):
                cols = slice(hh * DH, (hh + 1) * DH)
                qh = q_scr[b, :, cols]
                s = jnp.concatenate([
                    lax.dot_general(
                        qh, k_full[b, g, :, cols],
                        (((1,), (1,)), ((), ())),
                        preferred_element_type=jnp.float32)
                    for g in range(N_DEV)
                ], axis=1) * 0.125
                m = jnp.max(s, axis=-1, keepdims=True)
                w = jnp.exp(s - m)
                w = w / jnp.sum(w, axis=-1, keepdims=True)
                ctx_heads.append(sum(
                    jnp.dot(w[:, g * S_LOC:(g + 1) * S_LOC],
                            v_full[b, g, :, cols],
                            preferred_element_type=jnp.float32)
                    for g in range(N_DEV)
                ))
            ctx = jnp.concatenate(ctx_heads, axis=1)
            out_ref[b] = jnp.dot(
                ctx, wo_ref[...], preferred_element_type=jnp.float32)

    return pl.pallas_call(
        body,
        out_shape=jax.ShapeDtypeStruct((B, S_LOC, D), jnp.float32),
        in_specs=[pl.BlockSpec(memory_space=pltpu.VMEM)] * 5,
        out_specs=pl.BlockSpec(memory_space=pltpu.VMEM),
        scratch_shapes=[
            pltpu.VMEM((B, S_LOC, HD), jnp.float32),
            pltpu.VMEM((B, N_DEV, S_LOC, HD), jnp.float32),
            pltpu.VMEM((B, N_DEV, S_LOC, HD), jnp.float32),
            pltpu.SemaphoreType.DMA((N_XFER,)),
            pltpu.SemaphoreType.DMA((N_XFER,)),
        ],
        compiler_params=pltpu.CompilerParams(collective_id=0),
    )(x, Wq, Wk, Wv, Wo)


# device time: 21677 ns/iter; 1.6708x vs baseline; 1.1902x over previous
import jax
import jax.numpy as jnp
from jax import lax
from jax.experimental import pallas as pl
from jax.experimental.pallas import tpu as pltpu

N_DEV = 4
B = 2
S_LOC = 128
D = 512
HQ = 4
DH = 64
HD = HQ * DH
N_XFER = 12


def kernel(x, Wq, Wk, Wv, Wo):
    def body(x_ref, wq_ref, wk_ref, wv_ref, wo_ref, out_ref,
             q_scr, k_full, v_full, send_sems, recv_sems):
        my_pos = lax.axis_index("i")
        left = lax.rem(my_pos - 1 + N_DEV, N_DEV)
        right = lax.rem(my_pos + 1, N_DEV)
        far = lax.rem(my_pos + 2, N_DEV)

        barrier_sem = pltpu.get_barrier_semaphore()
        for nbr in [left, right]:
            pl.semaphore_signal(
                barrier_sem, inc=1,
                device_id=(nbr,), device_id_type=pl.DeviceIdType.MESH,
            )
        pl.semaphore_wait(barrier_sem, 2)

        def rdma(src, dst, idx, tgt):
            return pltpu.make_async_remote_copy(
                src_ref=src, dst_ref=dst,
                send_sem=send_sems.at[idx], recv_sem=recv_sems.at[idx],
                device_id=(tgt,), device_id_type=pl.DeviceIdType.MESH,
            )

        pos = (my_pos * S_LOC).astype(jnp.float32) + lax.broadcasted_iota(
            jnp.int32, (S_LOC, HD), 0).astype(jnp.float32)
        lane = lax.broadcasted_iota(jnp.int32, (S_LOC, HD), 1)
        d = lane % DH
        expo = (d - (d % 2)).astype(jnp.float32) / DH
        inv = jnp.exp(-jnp.log(10000.0) * expo)
        ang = pos * inv
        cos = jnp.cos(ang)
        sin = jnp.sin(ang)

        r = lax.broadcasted_iota(jnp.int32, (HD, HD), 0)
        c = lax.broadcasted_iota(jnp.int32, (HD, HD), 1)
        P = (((c == r + 1) & (r % 2 == 0)).astype(jnp.float32)
             - ((c == r - 1) & (r % 2 == 1)).astype(jnp.float32))

        bf16 = jnp.bfloat16
        Pb = P.astype(bf16)
        wk_b = wk_ref[...].astype(bf16)
        wv_b = wv_ref[...].astype(bf16)
        wq_b = wq_ref[...].astype(bf16)
        wo_b = wo_ref[...].astype(bf16)

        sends = []
        for b in range(B):
            xb = x_ref[b].astype(bf16)
            kp = jnp.dot(xb, wk_b, preferred_element_type=jnp.float32)
            kb = kp * cos + jnp.dot(
                kp.astype(bf16), Pb, preferred_element_type=jnp.float32) * sin
            vb = jnp.dot(xb, wv_b, preferred_element_type=jnp.float32)
            k_full[b, my_pos] = kb.astype(bf16)
            v_full[b, my_pos] = vb.astype(bf16)
            for s in (
                rdma(k_full.at[b, my_pos], k_full.at[b, my_pos], 2 * b + 0, right),
                rdma(v_full.at[b, my_pos], v_full.at[b, my_pos], 2 * b + 1, right),
                rdma(k_full.at[b, my_pos], k_full.at[b, my_pos], 2 * b + 4, left),
                rdma(v_full.at[b, my_pos], v_full.at[b, my_pos], 2 * b + 5, left),
            ):
                s.start()
                sends.append(s)

        for b in range(B):
            qp = jnp.dot(x_ref[b].astype(bf16), wq_b,
                         preferred_element_type=jnp.float32)
            q_scr[b] = (qp * cos + jnp.dot(
                qp.astype(bf16), Pb,
                preferred_element_type=jnp.float32) * sin).astype(bf16)

        def recv(dst, idx):
            return rdma(dst, dst, idx, right)

        recvs = [
            recv(k_full.at[0, left], 0), recv(v_full.at[0, left], 1),
            recv(k_full.at[1, left], 2), recv(v_full.at[1, left], 3),
            recv(k_full.at[0, right], 4), recv(v_full.at[0, right], 5),
            recv(k_full.at[1, right], 6), recv(v_full.at[1, right], 7),
            recv(k_full.at[0, far], 8), recv(v_full.at[0, far], 9),
            recv(k_full.at[1, far], 10), recv(v_full.at[1, far], 11),
        ]

        recvs[0].wait_recv()
        recvs[1].wait_recv()
        for s in (rdma(k_full.at[0, left], k_full.at[0, left], 8, right),
                  rdma(v_full.at[0, left], v_full.at[0, left], 9, right)):
            s.start()
            sends.append(s)
        recvs[6].wait_recv()
        recvs[7].wait_recv()
        for s in (rdma(k_full.at[1, right], k_full.at[1, right], 10, left),
                  rdma(v_full.at[1, right], v_full.at[1, right], 11, left)):
            s.start()
            sends.append(s)

        for i in (2, 3, 4, 5, 8, 9, 10, 11):
            recvs[i].wait_recv()
        for s in sends:
            s.wait_send()

        for b in range(B):
            ctx_heads = []
            for hh in range(HQ):
                cols = slice(hh * DH, (hh + 1) * DH)
                qh = q_scr[b, :, cols]
                s = jnp.concatenate([
                    lax.dot_general(
                        qh, k_full[b, g, :, cols],
                        (((1,), (1,)), ((), ())),
                        preferred_element_type=jnp.float32)
                    for g in range(N_DEV)
                ], axis=1) * 0.125
                m = jnp.max(s, axis=-1, keepdims=True)
                w = jnp.exp(s - m)
                w = (w / jnp.sum(w, axis=-1, keepdims=True)).astype(bf16)
                ctx_heads.append(sum(
                    jnp.dot(w[:, g * S_LOC:(g + 1) * S_LOC],
                            v_full[b, g, :, cols],
                            preferred_element_type=jnp.float32)
                    for g in range(N_DEV)
                ))
            ctx = jnp.concatenate(ctx_heads, axis=1)
            out_ref[b] = jnp.dot(
                ctx.astype(bf16), wo_b, preferred_element_type=jnp.float32)

    return pl.pallas_call(
        body,
        out_shape=jax.ShapeDtypeStruct((B, S_LOC, D), jnp.float32),
        in_specs=[pl.BlockSpec(memory_space=pltpu.VMEM)] * 5,
        out_specs=pl.BlockSpec(memory_space=pltpu.VMEM),
        scratch_shapes=[
            pltpu.VMEM((B, S_LOC, HD), jnp.bfloat16),
            pltpu.VMEM((B, N_DEV, S_LOC, HD), jnp.bfloat16),
            pltpu.VMEM((B, N_DEV, S_LOC, HD), jnp.bfloat16),
            pltpu.SemaphoreType.DMA((N_XFER,)),
            pltpu.SemaphoreType.DMA((N_XFER,)),
        ],
        compiler_params=pltpu.CompilerParams(collective_id=0),
    )(x, Wq, Wk, Wv, Wo)


# device time: 16701 ns/iter; 2.1686x vs baseline; 1.2979x over previous
import jax
import jax.numpy as jnp
from jax import lax
from jax.experimental import pallas as pl
from jax.experimental.pallas import tpu as pltpu

N_DEV = 4
B = 2
S_LOC = 128
S_GLOB = S_LOC * N_DEV
D = 512
HQ = 4
DH = 64
HD = HQ * DH


def kernel(x, Wq, Wk, Wv, Wo):
    def body(x_ref, wq_ref, wk_ref, wv_ref, wo_ref, out_ref,
             k_all, v_all, send_sems, recv_sems):
        my_pos = lax.axis_index("i")
        left = lax.rem(my_pos - 1 + N_DEV, N_DEV)
        right = lax.rem(my_pos + 1, N_DEV)

        barrier_sem = pltpu.get_barrier_semaphore()
        for nbr in [left, right]:
            pl.semaphore_signal(
                barrier_sem, inc=1,
                device_id=(nbr,), device_id_type=pl.DeviceIdType.MESH,
            )

        def rdma(src, dst, sem_idx, tgt):
            return pltpu.make_async_remote_copy(
                src_ref=src, dst_ref=dst,
                send_sem=send_sems.at[sem_idx], recv_sem=recv_sems.at[sem_idx],
                device_id=(tgt,), device_id_type=pl.DeviceIdType.MESH,
            )

        pos = (my_pos * S_LOC).astype(jnp.float32) + lax.broadcasted_iota(
            jnp.int32, (S_LOC, HD), 0).astype(jnp.float32)
        lane = lax.broadcasted_iota(jnp.int32, (S_LOC, HD), 1)
        dd = lane % DH
        expo = (dd - (dd % 2)).astype(jnp.float32) / DH
        inv = jnp.exp(-jnp.log(10000.0) * expo)
        ang = pos * inv
        cos = jnp.cos(ang)
        sin = jnp.sin(ang)

        r = lax.broadcasted_iota(jnp.int32, (HD, HD), 0)
        c = lax.broadcasted_iota(jnp.int32, (HD, HD), 1)
        bf16 = jnp.bfloat16
        P = (((c == r + 1) & (r % 2 == 0)).astype(bf16)
             - ((c == r - 1) & (r % 2 == 1)).astype(bf16))

        wk_b = wk_ref[...].astype(bf16)
        wv_b = wv_ref[...].astype(bf16)
        wq_b = wq_ref[...].astype(bf16)
        wo_b = wo_ref[...].astype(bf16)
        xb = [x_ref[b].astype(bf16) for b in range(B)]


        sends = []

        def send(s):
            s.start()
            sends.append(s)

        def kproj(b):
            kp = jnp.dot(xb[b], wk_b, preferred_element_type=jnp.float32)
            kb = kp * cos + jnp.dot(
                kp.astype(bf16), P, preferred_element_type=jnp.float32) * sin
            k_all[0, b] = kb.astype(bf16)

        def vproj(b):
            vp = jnp.dot(xb[b], wv_b, preferred_element_type=jnp.float32)
            v_all[0, b] = vp.astype(bf16)

        kproj(0)
        kproj(1)
        pl.semaphore_wait(barrier_sem, 2)
        send(rdma(k_all.at[0, 1], k_all.at[1, 1], 0, right))
        send(rdma(k_all.at[0, 0], k_all.at[2, 0], 1, left))
        vproj(1)
        send(rdma(v_all.at[0, 1], v_all.at[1, 1], 2, right))
        vproj(0)
        send(rdma(v_all.at[0, 0], v_all.at[2, 0], 3, left))
        send(rdma(k_all.at[0, 0], k_all.at[1, 0], 4, right))
        send(rdma(k_all.at[0, 1], k_all.at[2, 1], 5, left))
        send(rdma(v_all.at[0, 0], v_all.at[1, 0], 6, right))
        send(rdma(v_all.at[0, 1], v_all.at[2, 1], 7, left))

        q = []
        for b in range(B):
            qp = jnp.dot(xb[b], wq_b, preferred_element_type=jnp.float32)
            qr = qp * cos + jnp.dot(
                qp.astype(bf16), P, preferred_element_type=jnp.float32) * sin
            q.append((qr * 0.125).astype(bf16))

        rv = [
            rdma(k_all.at[1, 1], k_all.at[1, 1], 0, right),
            rdma(k_all.at[2, 0], k_all.at[2, 0], 1, right),
            rdma(v_all.at[1, 1], v_all.at[1, 1], 2, right),
            rdma(v_all.at[2, 0], v_all.at[2, 0], 3, right),
            rdma(k_all.at[1, 0], k_all.at[1, 0], 4, right),
            rdma(k_all.at[2, 1], k_all.at[2, 1], 5, right),
            rdma(v_all.at[1, 0], v_all.at[1, 0], 6, right),
            rdma(v_all.at[2, 1], v_all.at[2, 1], 7, right),
            rdma(k_all.at[3, 1], k_all.at[3, 1], 8, right),
            rdma(k_all.at[3, 0], k_all.at[3, 0], 9, right),
            rdma(v_all.at[3, 1], v_all.at[3, 1], 10, right),
            rdma(v_all.at[3, 0], v_all.at[3, 0], 11, right),
        ]

        l_st = [[[] for _ in range(HQ)] for _ in range(B)]
        p_st = [[{} for _ in range(HQ)] for _ in range(B)]
        pc_st = [[[] for _ in range(HQ)] for _ in range(B)]

        def do_scores(slot, bs=(0, 1)):
            for b in bs:
                for hh in range(HQ):
                    cols = slice(hh * DH, (hh + 1) * DH)
                    s_blk = lax.dot_general(
                        q[b][:, cols], k_all[slot, b, :, cols],
                        (((1,), (1,)), ((), ())),
                        preferred_element_type=jnp.float32)
                    p = jnp.exp(s_blk)
                    l_st[b][hh].append(jnp.sum(p, axis=-1, keepdims=True))
                    p_st[b][hh][slot] = p.astype(bf16)

        def do_ctx(slot, bs=(0, 1)):
            for b in bs:
                for hh in range(HQ):
                    cols = slice(hh * DH, (hh + 1) * DH)
                    pc_st[b][hh].append(jnp.dot(
                        p_st[b][hh][slot],
                        v_all[slot, b, :, cols],
                        preferred_element_type=jnp.float32))

        do_scores(0)
        do_ctx(0)

        rv[0].wait_recv()
        send(rdma(k_all.at[1, 1], k_all.at[3, 1], 8, right))
        do_scores(1, bs=(1,))
        rv[1].wait_recv()
        send(rdma(k_all.at[2, 0], k_all.at[3, 0], 9, left))
        do_scores(2, bs=(0,))
        rv[2].wait_recv()
        send(rdma(v_all.at[1, 1], v_all.at[3, 1], 10, right))
        do_ctx(1, bs=(1,))
        rv[3].wait_recv()
        send(rdma(v_all.at[2, 0], v_all.at[3, 0], 11, left))
        do_ctx(2, bs=(0,))
        rv[4].wait_recv()
        do_scores(1, bs=(0,))
        rv[5].wait_recv()
        do_scores(2, bs=(1,))
        rv[6].wait_recv()
        do_ctx(1, bs=(0,))
        rv[7].wait_recv()
        do_ctx(2, bs=(1,))

        rv[8].wait_recv()
        do_scores(3, bs=(1,))
        rv[9].wait_recv()
        do_scores(3, bs=(0,))
        rv[10].wait_recv()
        do_ctx(3, bs=(1,))
        rv[11].wait_recv()
        do_ctx(3, bs=(0,))
        for s in sends:
            s.wait_send()

        for b in range(B):
            ctx_heads = []
            for hh in range(HQ):
                L = sum(l_st[b][hh])
                acc = sum(pc_st[b][hh])
                ctx_heads.append(acc / L)
            ctx = jnp.concatenate(ctx_heads, axis=1)
            out_ref[b] = jnp.dot(
                ctx.astype(bf16), wo_b, preferred_element_type=jnp.float32)

    return pl.pallas_call(
        body,
        out_shape=jax.ShapeDtypeStruct((B, S_LOC, D), jnp.float32),
        in_specs=[pl.BlockSpec(memory_space=pltpu.VMEM)] * 5,
        out_specs=pl.BlockSpec(memory_space=pltpu.VMEM),
        scratch_shapes=[
            pltpu.VMEM((N_DEV, B, S_LOC, HD), jnp.bfloat16),
            pltpu.VMEM((N_DEV, B, S_LOC, HD), jnp.bfloat16),
            pltpu.SemaphoreType.DMA((12,)),
            pltpu.SemaphoreType.DMA((12,)),
        ],
        compiler_params=pltpu.CompilerParams(collective_id=0),
    )(x, Wq, Wk, Wv, Wo)
